# baseline (speedup 1.0000x reference)
import functools

import jax
import jax.numpy as jnp
from jax import lax
from jax.experimental import pallas as pl
from jax.experimental.pallas import tpu as pltpu
from jax.experimental.pallas import tpu_sc as plsc

_NUM_CORES = 2
_NUM_SUBCORES = 16
_NUM_WORKERS = _NUM_CORES * _NUM_SUBCORES
_CHUNK = 128


def _gather_rows(table, idx_flat, n_per_w, n_chunks, kp):
    mesh = plsc.VectorSubcoreMesh(core_axis_name="c", subcore_axis_name="s")

    @functools.partial(
        pl.kernel,
        mesh=mesh,
        compiler_params=pltpu.CompilerParams(use_tc_tiling_on_sc=False),
        out_type=jax.ShapeDtypeStruct((idx_flat.shape[0], kp), table.dtype),
        scratch_types=[
            pltpu.VMEM((n_chunks, _CHUNK), jnp.int32),
            pltpu.VMEM((n_per_w, kp), table.dtype),
            pltpu.SemaphoreType.DMA,
        ],
    )
    def run(table_hbm, idx_hbm, out_hbm, idx_v, rows_v, sem):
        wid = lax.axis_index("s") * _NUM_CORES + lax.axis_index("c")
        base = wid * n_per_w
        for j in range(n_chunks):
            pltpu.sync_copy(idx_hbm.at[pl.ds(base + j * _CHUNK, _CHUNK)],
                            idx_v.at[j])
        copies = [
            pltpu.async_copy(
                table_hbm.at[idx_v.at[j]],
                rows_v.at[pl.ds(j * _CHUNK, _CHUNK)],
                sem,
            )
            for j in range(n_chunks)
        ]
        for c in copies:
            c.wait()
        pltpu.sync_copy(rows_v, out_hbm.at[pl.ds(base, n_per_w)])

    return run(table, idx_flat)


def kernel(alias_indices, alias2entity_table):
    b, m = alias_indices.shape
    k = alias2entity_table.shape[1]
    kp = (k + 7) // 8 * 8
    n = b * m
    assert n % (_NUM_WORKERS * _CHUNK) == 0
    n_per_w = n // _NUM_WORKERS
    n_chunks = n_per_w // _CHUNK
    table_p = alias2entity_table
    if kp != k:
        table_p = jnp.pad(alias2entity_table, ((0, 0), (0, kp - k)))
    idx_flat = alias_indices.reshape(n).astype(jnp.int32)
    out = _gather_rows(table_p, idx_flat, n_per_w, n_chunks, kp)
    return out[:, :k].reshape(b, m, k)
